# Initial kernel scaffold; baseline (speedup 1.0000x reference)
import jax
import jax.numpy as jnp
from jax.experimental import pallas as pl


def kernel(feat, edge_index, edge_dist, W1, b1, Wsrc, bsrc, Wdst, bdst, Watt, batt, belta):
    raise NotImplementedError("write your pallas kernel here")


# speedup vs baseline: 8.4539x; 8.4539x over previous
import functools

import jax
import jax.numpy as jnp
from jax import lax
from jax.experimental import pallas as pl
from jax.experimental.pallas import tpu as pltpu
from jax.experimental.pallas import tpu_sc as plsc

_N = 10000
_E = 320000
_IN_F = 128
_OUT_F = 128
_EMB = 32

_NC = 2
_NS = 16
_L = 16
_NW = _NC * _NS
_EPW = _E // _NW
_KC = 80
_NCHUNK = _EPW // _KC
_NPAD = 10240
_RPT = _NPAD // _NS

_ROW_BLK = 1000


def _proj_body(belta_ref, feat_ref, w1t_ref, b1_ref, wst_ref, bs_ref,
               wdt_ref, bd_ref, wat_ref, ba_ref,
               fsrc_ref, semb_ref, demb_ref, eatt_ref):
    f = feat_ref[...]
    b = belta_ref[0]
    fsrc_ref[...] = jnp.dot(f, w1t_ref[...],
                            preferred_element_type=jnp.float32) + b1_ref[...]
    semb_ref[...] = (jnp.dot(f, wst_ref[...],
                             preferred_element_type=jnp.float32)
                     + bs_ref[...]) * b
    demb_ref[...] = jnp.dot(f, wdt_ref[...],
                            preferred_element_type=jnp.float32) + bd_ref[...]
    eatt_ref[...] = jnp.dot(jnp.maximum(f, 0.0), wat_ref[...],
                            preferred_element_type=jnp.float32) + ba_ref[...]


def _projections(feat, w1t, b1, wst, bs, wdt, bd, wat, ba, belta):
    nblk = _N // _ROW_BLK
    full = lambda *_: (0, 0)
    row = lambda i: (i, 0)
    return pl.pallas_call(
        _proj_body,
        grid=(nblk,),
        in_specs=[
            pl.BlockSpec(memory_space=pltpu.SMEM),
            pl.BlockSpec((_ROW_BLK, _IN_F), row),
            pl.BlockSpec((_IN_F, _OUT_F), full),
            pl.BlockSpec((1, _OUT_F), full),
            pl.BlockSpec((_IN_F, _EMB), full),
            pl.BlockSpec((1, _EMB), full),
            pl.BlockSpec((_IN_F, _EMB), full),
            pl.BlockSpec((1, _EMB), full),
            pl.BlockSpec((_IN_F, _OUT_F), full),
            pl.BlockSpec((1, _OUT_F), full),
        ],
        out_specs=[
            pl.BlockSpec((_ROW_BLK, _OUT_F), row),
            pl.BlockSpec((_ROW_BLK, _EMB), row),
            pl.BlockSpec((_ROW_BLK, _EMB), row),
            pl.BlockSpec((_ROW_BLK, _OUT_F), row),
        ],
        out_shape=[
            jax.ShapeDtypeStruct((_N, _OUT_F), jnp.float32),
            jax.ShapeDtypeStruct((_N, _EMB), jnp.float32),
            jax.ShapeDtypeStruct((_N, _EMB), jnp.float32),
            jax.ShapeDtypeStruct((_N, _OUT_F), jnp.float32),
        ],
    )(belta, feat, w1t, b1, wst, bs, wdt, bd, wat, ba)


def _edge_body(src_hbm, dst_hbm, dist_hbm, semb_hbm, demb_hbm, fsrc_hbm,
               zeros_hbm, out_hbm,
               sidx, didx, distv, srows, drows, frows, ft_sh,
               sem_a, sem_b):
    cid = lax.axis_index("c")
    sid = lax.axis_index("s")
    wid = sid * _NC + cid

    pltpu.sync_copy(zeros_hbm, ft_sh.at[pl.ds(sid * _RPT, _RPT)])
    plsc.subcore_barrier()

    base0 = wid * _EPW

    def chunk(c, carry):
        base = base0 + c * _KC
        d1 = pltpu.async_copy(src_hbm.at[pl.ds(base, _KC)], sidx, sem_a)
        d2 = pltpu.async_copy(dst_hbm.at[pl.ds(base, _KC)], didx, sem_a)
        d3 = pltpu.async_copy(dist_hbm.at[pl.ds(base, _KC)], distv, sem_a)
        d1.wait(); d2.wait(); d3.wait()
        g1 = pltpu.async_copy(semb_hbm.at[sidx], srows, sem_b)
        g2 = pltpu.async_copy(demb_hbm.at[didx], drows, sem_b)
        g3 = pltpu.async_copy(fsrc_hbm.at[sidx], frows, sem_b)
        g1.wait(); g2.wait(); g3.wait()

        for g in range(_KC // _L):
            invd = 1.0 / distv[pl.ds(g * _L, _L)]
            for i in range(_L):
                e = g * _L + i
                s0 = srows[e, pl.ds(0, _L)]
                s1 = srows[e, pl.ds(_L, _L)]
                d0 = drows[e, pl.ds(0, _L)]
                d1 = drows[e, pl.ds(_L, _L)]
                xe = jnp.sum(s0 * d0 + s1 * d1)
                w = jnp.broadcast_to(xe, (_L,)) * jnp.broadcast_to(invd[i], (_L,))
                for j in range(_OUT_F // _L):
                    sl = pl.ds(j * _L, _L)
                    frows[e, sl] = frows[e, sl] * w

        pltpu.sync_copy(frows, ft_sh.at[didx], add=True)
        return carry

    lax.fori_loop(0, _NCHUNK, chunk, 0)

    plsc.subcore_barrier()
    off = (cid * _NS + sid) * _RPT
    pltpu.sync_copy(ft_sh.at[pl.ds(sid * _RPT, _RPT)],
                    out_hbm.at[pl.ds(off, _RPT)])


def _edge_aggregate(src, dst, dist, semb, demb, fsrc):
    zeros = jnp.zeros((_RPT, _OUT_F), jnp.float32)
    mesh = plsc.VectorSubcoreMesh(core_axis_name="c", subcore_axis_name="s")
    run = pl.kernel(
        _edge_body,
        out_type=jax.ShapeDtypeStruct((_NC * _NPAD, _OUT_F), jnp.float32),
        mesh=mesh,
        compiler_params=pltpu.CompilerParams(needs_layout_passes=False,
                                             use_tc_tiling_on_sc=False),
        scratch_types=[
            pltpu.VMEM((_KC,), jnp.int32),
            pltpu.VMEM((_KC,), jnp.int32),
            pltpu.VMEM((_KC,), jnp.float32),
            pltpu.VMEM((_KC, _EMB), jnp.float32),
            pltpu.VMEM((_KC, _EMB), jnp.float32),
            pltpu.VMEM((_KC, _OUT_F), jnp.float32),
            pltpu.VMEM_SHARED((_NPAD, _OUT_F), jnp.float32),
            pltpu.SemaphoreType.DMA,
            pltpu.SemaphoreType.DMA,
        ],
    )
    return run(src, dst, dist, semb, demb, fsrc, zeros)


def _final_body(eatt_ref, ft_ref, out_ref):
    x = eatt_ref[...] * (ft_ref[0] + ft_ref[1])
    out_ref[...] = jnp.where(x > 0.0, x, jnp.exp(x) - 1.0)


def _finalize(eatt, ft2):
    nblk = _N // _ROW_BLK
    return pl.pallas_call(
        _final_body,
        grid=(nblk,),
        in_specs=[
            pl.BlockSpec((_ROW_BLK, _OUT_F), lambda i: (i, 0)),
            pl.BlockSpec((2, _ROW_BLK, _OUT_F), lambda i: (0, i, 0)),
        ],
        out_specs=pl.BlockSpec((_ROW_BLK, _OUT_F), lambda i: (i, 0)),
        out_shape=jax.ShapeDtypeStruct((_N, _OUT_F), jnp.float32),
    )(eatt, ft2)


def kernel(feat, edge_index, edge_dist, W1, b1, Wsrc, bsrc, Wdst, bdst,
           Watt, batt, belta):
    src = edge_index[0]
    dst = edge_index[1]
    fsrc, semb, demb, eatt = _projections(
        feat, W1.T, b1[None, :], Wsrc.T, bsrc[None, :], Wdst.T, bdst[None, :],
        Watt.T, batt[None, :], belta)
    ft = _edge_aggregate(src, dst, edge_dist, semb, demb, fsrc)
    ft2 = ft.reshape(_NC, _NPAD, _OUT_F)
    return _finalize(eatt, ft2)
